# Initial kernel scaffold; baseline (speedup 1.0000x reference)
import jax
import jax.numpy as jnp
from jax.experimental import pallas as pl


def kernel(x, weight):
    raise NotImplementedError("write your pallas kernel here")


# speedup vs baseline: 1.0425x; 1.0425x over previous
import jax
import jax.numpy as jnp
from jax.experimental import pallas as pl
from jax.experimental.pallas import tpu as pltpu
from jax.experimental.pallas import tpu_sc as plsc

_WINDOW = 128


def kernel(x, weight):
    B, H = x.shape
    N, D = weight.shape
    num_indices = B * H
    assert num_indices % _WINDOW == 0

    idx = x.reshape(1, num_indices).astype(jnp.int32)
    mesh = plsc.VectorSubcoreMesh(core_axis_name="core", subcore_axis_name="subcore")

    @pl.kernel(
        out_type=jax.ShapeDtypeStruct((num_indices, D), weight.dtype),
        mesh=mesh,
        compiler_params=pltpu.CompilerParams(use_tc_tiling_on_sc=False),
    )
    def gather_kernel(w_hbm, i_hbm, o_hbm):
        def body(i_vmem, o_vmem):
            pltpu.sync_copy(w_hbm.at[i_vmem.at[0]], o_vmem)

        pltpu.emit_pipeline(
            body,
            grid=(num_indices // _WINDOW,),
            in_specs=[pl.BlockSpec((1, _WINDOW), index_map=lambda i: (0, i))],
            out_specs=[pl.BlockSpec((_WINDOW, D), index_map=lambda i: (i, 0))],
            core_axis_name=("core", "subcore"),
            dimension_semantics=(pltpu.PARALLEL,),
        )(i_hbm, o_hbm)

    out = gather_kernel(weight, idx)
    return out.reshape(B, H, D)


# speedup vs baseline: 1.1110x; 1.0657x over previous
import jax
import jax.numpy as jnp
from jax import lax
from jax.experimental import pallas as pl
from jax.experimental.pallas import tpu as pltpu
from jax.experimental.pallas import tpu_sc as plsc

_W = 128
_K = 8
_NTILES = 32


def kernel(x, weight):
    B, H = x.shape
    N, D = weight.shape
    num_indices = B * H
    assert num_indices % (_NTILES * _W) == 0
    wpt = num_indices // (_NTILES * _W)
    assert wpt % _K == 0

    idx = x.reshape(_NTILES, wpt, _W).astype(jnp.int32)
    mesh = plsc.VectorSubcoreMesh(core_axis_name="core", subcore_axis_name="subcore")

    @pl.kernel(
        out_type=jax.ShapeDtypeStruct((num_indices, D), weight.dtype),
        mesh=mesh,
        compiler_params=pltpu.CompilerParams(use_tc_tiling_on_sc=False),
        scratch_types=[
            pltpu.VMEM((wpt, _W), jnp.int32),
            pltpu.VMEM((_K, _W, D), jnp.float32),
            pltpu.SemaphoreType.DMA,
            pltpu.SemaphoreType.DMA,
            pltpu.SemaphoreType.DMA,
        ],
    )
    def gather_kernel(w_hbm, i_hbm, o_hbm, idx_v, bufs, isem, gsem, osem):
        wid = lax.axis_index("core") * 16 + lax.axis_index("subcore")
        pltpu.async_copy(i_hbm.at[wid], idx_v, isem).wait()
        tile_row0 = wid * (wpt * _W)

        @pl.loop(0, wpt, step=_K)
        def _(w0):
            gathers = []
            for s in range(_K):
                gathers.append(
                    pltpu.async_copy(w_hbm.at[idx_v.at[w0 + s]], bufs.at[s], gsem)
                )
            stores = []
            for s in range(_K):
                gathers[s].wait()
                row = tile_row0 + (w0 + s) * _W
                stores.append(
                    pltpu.async_copy(bufs.at[s], o_hbm.at[pl.ds(row, _W)], osem)
                )
            for s in range(_K):
                stores[s].wait()

    out = gather_kernel(weight, idx)
    return out.reshape(B, H, D)


# speedup vs baseline: 1.3328x; 1.1997x over previous
import jax
import jax.numpy as jnp
from jax import lax
from jax.experimental import pallas as pl
from jax.experimental.pallas import tpu as pltpu
from jax.experimental.pallas import tpu_sc as plsc

_W = 128
_K = 4
_NTILES = 32
_L = 16


def kernel(x, weight):
    B, H = x.shape
    N, D = weight.shape
    num_windows = B * H // _W
    assert num_windows % (_NTILES * _K) == 0 and B % _W == 0
    wpt = num_windows // _NTILES
    nbt = B // _W

    idx_t = x.T.astype(jnp.int32)
    mesh = plsc.VectorSubcoreMesh(core_axis_name="core", subcore_axis_name="subcore")

    @pl.kernel(
        out_type=jax.ShapeDtypeStruct((H, D, B), weight.dtype),
        mesh=mesh,
        compiler_params=pltpu.CompilerParams(
            use_tc_tiling_on_sc=False, needs_layout_passes=False
        ),
        scratch_types=[
            pltpu.VMEM((_K, _W), jnp.int32),
            pltpu.VMEM((_K, _W, D), jnp.float32),
            pltpu.VMEM((_K, D, _W), jnp.float32),
            pltpu.SemaphoreType.DMA,
            pltpu.SemaphoreType.DMA,
            pltpu.SemaphoreType.DMA,
        ],
    )
    def gather_kernel(w_hbm, i_hbm, o_hbm, idx_v, g_v, t_v, isem, gsem, osem):
        wid = lax.axis_index("core") * 16 + lax.axis_index("subcore")
        w_base = wid * wpt

        def win_coords(wl):
            wg = w_base + wl
            return wg // nbt, lax.rem(wg, nbt)

        def fire_idx(wl, s):
            h, bt = win_coords(wl)
            return pltpu.async_copy(
                i_hbm.at[h, pl.ds(bt * _W, _W)], idx_v.at[s], isem
            )

        for s in range(_K):
            fire_idx(jnp.int32(s), s)

        @pl.loop(0, wpt, step=_K)
        def _(w0):
            gathers = []
            for s in range(_K):
                pltpu.make_async_copy(
                    i_hbm.at[0, pl.ds(0, _W)], idx_v.at[s], isem
                ).wait()
                gathers.append(
                    pltpu.async_copy(w_hbm.at[idx_v.at[s]], g_v.at[s], gsem)
                )
            stores = []
            for s in range(_K):
                wl = w0 + s
                h, bt = win_coords(wl)
                gathers[s].wait()
                for f in range(D):
                    col = jnp.full((_L,), f, jnp.int32)
                    for c in range(_W // _L):
                        rows = jax.lax.iota(jnp.int32, _L) + (c * _L)
                        t_v[s, f, pl.ds(c * _L, _L)] = plsc.load_gather(
                            g_v.at[s], [rows, col]
                        )
                stores.append(
                    pltpu.async_copy(
                        t_v.at[s], o_hbm.at[h, :, pl.ds(bt * _W, _W)], osem
                    )
                )
                nxt = wl + _K

                @pl.when(nxt < wpt)
                def _():
                    fire_idx(nxt, s)

            for s in range(_K):
                stores[s].wait()

    out = gather_kernel(weight, idx_t)
    return jnp.transpose(out, (2, 0, 1))


# speedup vs baseline: 1.9356x; 1.4523x over previous
import jax
import jax.numpy as jnp
from jax import lax
from jax.experimental import pallas as pl
from jax.experimental.pallas import tpu as pltpu
from jax.experimental.pallas import tpu_sc as plsc

_W = 128
_K = 8
_NTILES = 32


def kernel(x, weight):
    B, H = x.shape
    N, D = weight.shape
    num_windows = B * H // _W
    assert num_windows % (_NTILES * _K) == 0 and B % _W == 0
    wpt = num_windows // _NTILES
    nbt = B // _W

    idx_t = x.T.astype(jnp.int32)
    mesh = plsc.VectorSubcoreMesh(core_axis_name="core", subcore_axis_name="subcore")

    @pl.kernel(
        out_type=jax.ShapeDtypeStruct((H, B, D), weight.dtype),
        mesh=mesh,
        compiler_params=pltpu.CompilerParams(
            use_tc_tiling_on_sc=False, needs_layout_passes=False
        ),
        scratch_types=[
            pltpu.VMEM((_K, _W), jnp.int32),
            pltpu.VMEM((_K, _W, D), jnp.float32),
            pltpu.SemaphoreType.DMA,
            pltpu.SemaphoreType.DMA,
            pltpu.SemaphoreType.DMA,
        ],
    )
    def gather_kernel(w_hbm, i_hbm, o_hbm, idx_v, g_v, isem, gsem, osem):
        wid = lax.axis_index("core") * 16 + lax.axis_index("subcore")
        w_base = wid * wpt

        def win_coords(wl):
            wg = w_base + wl
            return wg // nbt, lax.rem(wg, nbt)

        def fire_idx(wl, s):
            h, bt = win_coords(wl)
            return pltpu.async_copy(
                i_hbm.at[h, pl.ds(bt * _W, _W)], idx_v.at[s], isem
            )

        for s in range(_K):
            fire_idx(jnp.int32(s), s)

        @pl.loop(0, wpt, step=_K)
        def _(w0):
            gathers = []
            for s in range(_K):
                pltpu.make_async_copy(
                    i_hbm.at[0, pl.ds(0, _W)], idx_v.at[s], isem
                ).wait()
                gathers.append(
                    pltpu.async_copy(w_hbm.at[idx_v.at[s]], g_v.at[s], gsem)
                )
            stores = []
            for s in range(_K):
                wl = w0 + s
                h, bt = win_coords(wl)
                gathers[s].wait()
                stores.append(
                    pltpu.async_copy(
                        g_v.at[s], o_hbm.at[h, pl.ds(bt * _W, _W), :], osem
                    )
                )
                nxt = wl + _K

                @pl.when(nxt < wpt)
                def _():
                    fire_idx(nxt, s)

            for s in range(_K):
                stores[s].wait()

    out = gather_kernel(weight, idx_t)
    return jnp.transpose(out, (1, 0, 2))
